# baseline (speedup 1.0000x reference)
import functools

import jax
import jax.numpy as jnp
from jax import lax
from jax.experimental import pallas as pl
from jax.experimental.pallas import tpu as pltpu
from jax.experimental.pallas import tpu_sc as plsc

B, S, D = 4, 8192, 1024
NC, NS = 2, 16
NW = NC * NS
S_PER_W = S // NW
CS = 32
N_CHUNK = S_PER_W // CS


def _sc_add(x2, emb):
    mesh = plsc.VectorSubcoreMesh(
        core_axis_name="c", subcore_axis_name="s", num_cores=NC, num_subcores=NS
    )

    @functools.partial(
        pl.kernel,
        out_type=jax.ShapeDtypeStruct((B * S, D), jnp.float32),
        mesh=mesh,
        scratch_types=[
            pltpu.VMEM((CS, D), jnp.float32),
            pltpu.VMEM((CS, D), jnp.float32),
            pltpu.VMEM((CS, D), jnp.float32),
            pltpu.SemaphoreType.DMA,
            pltpu.SemaphoreType.DMA,
            pltpu.SemaphoreType.DMA,
            pltpu.SemaphoreType.DMA,
        ],
    )
    def k(x_hbm, emb_hbm, out_hbm, xa, xb, emb_v, la, lb, sa, sb):
        wid = lax.axis_index("s") * NC + lax.axis_index("c")
        s_base = wid * S_PER_W
        bufs = (xa, xb)
        lsems = (la, lb)
        ssems = (sa, sb)

        def rows(c, b):
            return pl.ds(b * S + s_base + c * CS, CS)

        pltpu.async_copy(x_hbm.at[rows(0, 0)], xa, la)

        def chunk_body(c, carry):
            pltpu.sync_copy(emb_hbm.at[pl.ds(s_base + c * CS, CS)], emb_v)
            for b in range(B):
                p = b % 2
                q = 1 - p
                if b < B - 1:
                    if b == 0:
                        @pl.when(c > 0)
                        def _():
                            pltpu.make_async_copy(
                                bufs[q], out_hbm.at[rows(c - 1, 3)], ssems[q]
                            ).wait()
                    else:
                        pltpu.make_async_copy(
                            bufs[q], out_hbm.at[rows(c, b - 1)], ssems[q]
                        ).wait()
                    pltpu.async_copy(x_hbm.at[rows(c, b + 1)], bufs[q], lsems[q])
                else:
                    @pl.when(c < N_CHUNK - 1)
                    def _():
                        pltpu.make_async_copy(
                            bufs[q], out_hbm.at[rows(c, 2)], ssems[q]
                        ).wait()
                        pltpu.async_copy(x_hbm.at[rows(c + 1, 0)], bufs[q], lsems[q])
                pltpu.make_async_copy(x_hbm.at[rows(c, b)], bufs[p], lsems[p]).wait()
                xp = bufs[p]

                @plsc.parallel_loop(0, CS, step=1, unroll=2)
                def _(r):
                    for col in range(D // 16):
                        sl = pl.ds(col * 16, 16)
                        plsc.addupdate(xp.at[r, sl], emb_v[r, sl])

                pltpu.async_copy(bufs[p], out_hbm.at[rows(c, b)], ssems[p])
            return carry

        lax.fori_loop(0, N_CHUNK, chunk_body, 0)

        pltpu.make_async_copy(xa, out_hbm.at[rows(N_CHUNK - 1, 2)], sa).wait()
        pltpu.make_async_copy(xb, out_hbm.at[rows(N_CHUNK - 1, 3)], sb).wait()

    return k(x2, emb)


def kernel(x, emb):
    out = _sc_add(x.reshape(B * S, D), emb[:S])
    return out.reshape(B, S, D)


# speedup vs baseline: 1.8087x; 1.8087x over previous
import jax
import jax.numpy as jnp
from jax.experimental import pallas as pl
from jax.experimental.pallas import tpu as pltpu

B, S, D = 4, 8192, 1024
BS = 2048


def _add_kernel(x_ref, emb_ref, out_ref):
    out_ref[0] = x_ref[0] + emb_ref[...]


def kernel(x, emb):
    grid = (S // BS, B)
    return pl.pallas_call(
        _add_kernel,
        grid=grid,
        in_specs=[
            pl.BlockSpec((1, BS, D), lambda s, b: (b, s, 0)),
            pl.BlockSpec((BS, D), lambda s, b: (s, 0)),
        ],
        out_specs=pl.BlockSpec((1, BS, D), lambda s, b: (b, s, 0)),
        out_shape=jax.ShapeDtypeStruct((B, S, D), x.dtype),
        compiler_params=pltpu.CompilerParams(
            dimension_semantics=("parallel", "arbitrary"),
        ),
    )(x, emb[:S])
